# baseline (speedup 1.0000x reference)
import jax
import jax.numpy as jnp
from jax.experimental import pallas as pl

N = 8192
A = 2048
D_ATOM = 6
K = 16
SCALES = [1.0, 2.0, 3.0, 5.0, 10.0]
RADIUS = 9.0
IN_CH = 2 * len(SCALES) + D_ATOM
EMB = 64


def _lrelu(x):
    return jax.nn.leaky_relu(x, 0.2)


def _knn_idx(q, r, k, block=1024):
    nb = q.shape[0] // block
    qb = q.reshape(nb, block, q.shape[-1])
    def f(x):
        d2 = ((x[:, None, :] - r[None, :, :]) ** 2).sum(-1)
        return jax.lax.top_k(-d2, k)[1]
    return jax.lax.map(f, qb).reshape(q.shape[0], k)


def _tangent_vectors(n):
    x, y, z = n[:, 0], n[:, 1], n[:, 2]
    s = jnp.where(z >= 0, 1.0, -1.0)
    a = -1.0 / (s + z)
    b = x * y * a
    u = jnp.stack([1.0 + s * x * x * a, s * b, -s * x], axis=-1)
    v = jnp.stack([b, s + y * y * a, -y], axis=-1)
    return u, v


def _curvature_features(xyz, normals, idx):
    xj = xyz[idx]
    nj = normals[idx]
    dx = xj - xyz[:, None, :]
    d2 = (dx ** 2).sum(-1)
    feats = []
    eye = 0.01 * jnp.eye(2)[None]
    for s in SCALES:
        w = jnp.exp(-d2 / (2.0 * s * s))
        ns = (w[..., None] * nj).sum(1) + normals
        ns = ns / (jnp.linalg.norm(ns, axis=-1, keepdims=True) + 1e-8)
        u, v = _tangent_vectors(ns)
        uv = jnp.stack([u, v], axis=1)
        P = jnp.einsum('nij,nkj->nki', uv, dx)
        Q = jnp.einsum('nij,nkj->nki', uv, nj)
        PPt = jnp.einsum('nk,nki,nkj->nij', w, P, P)
        PQt = jnp.einsum('nk,nki,nkj->nij', w, P, Q)
        Sm = jnp.linalg.solve(PPt + eye, PQt)
        a_, b_, c_, d_ = Sm[:, 0, 0], Sm[:, 0, 1], Sm[:, 1, 0], Sm[:, 1, 1]
        feats.append(jnp.clip(a_ + d_, -1.0, 1.0))
        feats.append(jnp.clip(a_ * d_ - b_ * c_, -1.0, 1.0))
    return jnp.stack(feats, axis=-1)


def _atomnet(p, xyz, atom_xyz, atom_types, idx_aa, idx_pa):
    t = _lrelu(atom_types @ p['tt_W1'] + p['tt_b1']) @ p['tt_W2'] + p['tt_b2']
    d_aa = jnp.linalg.norm(atom_xyz[idx_aa] - atom_xyz[:, None, :], axis=-1, keepdims=True)
    y = t
    for i in range(3):
        h = jnp.concatenate([y[idx_aa], d_aa], axis=-1)
        m = _lrelu(h @ p['aa_W1'][i] + p['aa_b1'][i]) @ p['aa_W2'][i] + p['aa_b2'][i]
        y = y + jax.nn.relu(m.mean(axis=1))
    d_pa = jnp.linalg.norm(atom_xyz[idx_pa] - xyz[:, None, :], axis=-1, keepdims=True)
    fj = y[idx_pa]
    f = fj.mean(axis=1)
    for i in range(3):
        h = jnp.concatenate([fj, d_pa], axis=-1)
        m = _lrelu(h @ p['pa_W1'][i] + p['pa_b1'][i]) @ p['pa_W2'][i] + p['pa_b2'][i]
        f = f + jax.nn.relu(m.mean(axis=1))
    return f


def _load_mesh(xyz, normals, weights, idx):
    u, v = _tangent_vectors(normals)
    dx = xyz[idx] - xyz[:, None, :]
    d2 = (dx ** 2).sum(-1)
    w = jnp.exp(-d2 / (2.0 * RADIUS * RADIUS)) * weights[idx][..., 0]
    o = (w[..., None] * dx).sum(1)
    tu = (o * u).sum(-1)
    tv = (o * v).sum(-1)
    nrm = jnp.sqrt(tu * tu + tv * tv + 1e-8)
    tu = tu / nrm
    tv = tv / nrm
    u2 = tu[:, None] * u + tv[:, None] * v
    v2 = -tv[:, None] * u + tu[:, None] * v
    return jnp.stack([normals, u2, v2], axis=1)


def _dmasif_conv(lp, f, xyz, nuv, idx):
    f1 = f @ lp['in_W'] + lp['in_b']
    dx = xyz[idx] - xyz[:, None, :]
    d2 = (dx ** 2).sum(-1)
    ni = nuv[:, 0, :]
    cos = (ni[:, None, :] * ni[idx]).sum(-1)
    d2g = d2 * (2.0 - cos) ** 2
    w = jnp.exp(-d2g / (2.0 * RADIUS * RADIUS))
    X = jnp.einsum('nij,nkj->nki', nuv, dx) / RADIUS
    g = jax.nn.relu(X @ lp['f_W1'] + lp['f_b1']) @ lp['f_W2'] + lp['f_b2']
    out = (w[..., None] * g * f1[idx]).sum(1) / (w.sum(1, keepdims=True) + 1e-8)
    return out @ lp['out_W'] + lp['out_b']


def _mix_kernel(x_ref, w1_ref, b1_ref, w2_ref, b2_ref, wl_ref, bl_ref, f_ref, o_ref):
    x = x_ref[...]
    h = jnp.maximum(x @ w1_ref[...] + b1_ref[...], 0.0)
    o_ref[...] = (h @ w2_ref[...] + b2_ref[...]
                  + f_ref[...] @ wl_ref[...] + bl_ref[...])


def _mix(lp, xi, x, ich):
    return pl.pallas_call(
        _mix_kernel,
        out_shape=jax.ShapeDtypeStruct((x.shape[0], EMB), jnp.float32),
    )(xi, lp['mlp_W1'], lp['mlp_b1'][None], lp['mlp_W2'], lp['mlp_b2'][None],
      lp['lt_W'], lp['lt_b'][None], x)


def _conv_seg(layers, feats, xyz, nuv, idx):
    x = feats
    for lp in layers:
        xi = _dmasif_conv(lp, x, xyz, nuv, idx)
        x = _mix(lp, xi, x, x.shape[-1])
    return x


def _site_embed(p, xyz, normals, feats, idx):
    s = _lrelu(feats @ p['or_W1'] + p['or_b1']) @ p['or_W2'] + p['or_b2']
    nuv = _load_mesh(xyz, normals, s, idx)
    return _conv_seg(p['conv'], feats, xyz, nuv, idx)


def kernel(surface_xyz, surface_normals, atom_coords, atom_types, params):
    idx_ss = _knn_idx(surface_xyz, surface_xyz, K + 1)[:, 1:]
    idx_aa = _knn_idx(atom_coords, atom_coords, K + 1)[:, 1:]
    idx_pa = _knn_idx(surface_xyz, atom_coords, K)
    curv = _curvature_features(surface_xyz, surface_normals, idx_ss)
    chem = _atomnet(params['atomnet'], surface_xyz, atom_coords, atom_types, idx_aa, idx_pa)
    in_feats = jnp.concatenate([curv, chem], axis=1)
    out_feats = _site_embed(params['embed'], surface_xyz, surface_normals, in_feats, idx_ss)
    return in_feats, out_feats


# speedup vs baseline: 2.2460x; 2.2460x over previous
import jax
import jax.numpy as jnp
from jax.experimental import pallas as pl

N = 8192
A = 2048
D_ATOM = 6
K = 16
SCALES = [1.0, 2.0, 3.0, 5.0, 10.0]
RADIUS = 9.0
IN_CH = 2 * len(SCALES) + D_ATOM
EMB = 64


def _lrelu(x):
    return jax.nn.leaky_relu(x, 0.2)


def _knn_idx(q, r, k, block=1024):
    nb = q.shape[0] // block
    qb = q.reshape(nb, block, q.shape[-1])
    def f(x):
        d2 = ((x[:, None, :] - r[None, :, :]) ** 2).sum(-1)
        return jax.lax.top_k(-d2, k)[1]
    return jax.lax.map(f, qb).reshape(q.shape[0], k)


def _curv_kernel(xyzT_ref, nrmT_ref, xjT_ref, njT_ref,
                 curv_ref, d2_ref, cos_ref, dx_ref):
    xyzT = xyzT_ref[...]
    nrmT = nrmT_ref[...]
    xj = xjT_ref[...]
    nj = njT_ref[...]
    dx = xj - xyzT[:, None, :]
    d2 = (dx * dx).sum(0)
    dx_ref[...] = dx
    d2_ref[...] = d2
    cos_ref[...] = (nrmT[:, None, :] * nj).sum(0)
    rows = []
    for s in SCALES:
        w = jnp.exp(d2 * (-1.0 / (2.0 * s * s)))
        ns = (w[None] * nj).sum(1) + nrmT
        nslen = jnp.sqrt((ns * ns).sum(0, keepdims=True))
        ns = ns / (nslen + 1e-8)
        nx, ny, nz = ns[0:1], ns[1:2], ns[2:3]
        sg = jnp.where(nz >= 0, 1.0, -1.0)
        a = -1.0 / (sg + nz)
        b = nx * ny * a
        u = jnp.concatenate([1.0 + sg * nx * nx * a, sg * b, -sg * nx], axis=0)
        v = jnp.concatenate([b, sg + ny * ny * a, -ny], axis=0)
        P0 = (u[:, None, :] * dx).sum(0)
        P1 = (v[:, None, :] * dx).sum(0)
        Q0 = (u[:, None, :] * nj).sum(0)
        Q1 = (v[:, None, :] * nj).sum(0)
        wP0 = w * P0
        wP1 = w * P1
        ppt00 = (wP0 * P0).sum(0, keepdims=True) + 0.01
        ppt01 = (wP0 * P1).sum(0, keepdims=True)
        ppt11 = (wP1 * P1).sum(0, keepdims=True) + 0.01
        pqt00 = (wP0 * Q0).sum(0, keepdims=True)
        pqt01 = (wP0 * Q1).sum(0, keepdims=True)
        pqt10 = (wP1 * Q0).sum(0, keepdims=True)
        pqt11 = (wP1 * Q1).sum(0, keepdims=True)
        det = ppt00 * ppt11 - ppt01 * ppt01
        s00 = (ppt11 * pqt00 - ppt01 * pqt10) / det
        s01 = (ppt11 * pqt01 - ppt01 * pqt11) / det
        s10 = (ppt00 * pqt10 - ppt01 * pqt00) / det
        s11 = (ppt00 * pqt11 - ppt01 * pqt01) / det
        rows.append(jnp.clip(s00 + s11, -1.0, 1.0))
        rows.append(jnp.clip(s00 * s11 - s01 * s10, -1.0, 1.0))
    curv_ref[...] = jnp.concatenate(rows, axis=0)


def _curvature_pallas(xyz, normals, idx):
    xyzT = xyz.T
    nrmT = normals.T
    xjT = xyz[idx].transpose(2, 1, 0)
    njT = normals[idx].transpose(2, 1, 0)
    n = xyz.shape[0]
    curvT, d2T, cosT, dxT = pl.pallas_call(
        _curv_kernel,
        out_shape=[
            jax.ShapeDtypeStruct((2 * len(SCALES), n), jnp.float32),
            jax.ShapeDtypeStruct((K, n), jnp.float32),
            jax.ShapeDtypeStruct((K, n), jnp.float32),
            jax.ShapeDtypeStruct((3, K, n), jnp.float32),
        ],
    )(xyzT, nrmT, xjT, njT)
    return curvT, d2T, cosT, dxT


def _curvature_features(xyz, normals, idx):
    return _curvature_pallas(xyz, normals, idx)[0].T


def _tangent_vectors(n):
    x, y, z = n[:, 0], n[:, 1], n[:, 2]
    s = jnp.where(z >= 0, 1.0, -1.0)
    a = -1.0 / (s + z)
    b = x * y * a
    u = jnp.stack([1.0 + s * x * x * a, s * b, -s * x], axis=-1)
    v = jnp.stack([b, s + y * y * a, -y], axis=-1)
    return u, v


def _atomnet(p, xyz, atom_xyz, atom_types, idx_aa, idx_pa):
    t = _lrelu(atom_types @ p['tt_W1'] + p['tt_b1']) @ p['tt_W2'] + p['tt_b2']
    d_aa = jnp.linalg.norm(atom_xyz[idx_aa] - atom_xyz[:, None, :], axis=-1, keepdims=True)
    y = t
    for i in range(3):
        h = jnp.concatenate([y[idx_aa], d_aa], axis=-1)
        m = _lrelu(h @ p['aa_W1'][i] + p['aa_b1'][i]) @ p['aa_W2'][i] + p['aa_b2'][i]
        y = y + jax.nn.relu(m.mean(axis=1))
    d_pa = jnp.linalg.norm(atom_xyz[idx_pa] - xyz[:, None, :], axis=-1, keepdims=True)
    fj = y[idx_pa]
    f = fj.mean(axis=1)
    for i in range(3):
        h = jnp.concatenate([fj, d_pa], axis=-1)
        m = _lrelu(h @ p['pa_W1'][i] + p['pa_b1'][i]) @ p['pa_W2'][i] + p['pa_b2'][i]
        f = f + jax.nn.relu(m.mean(axis=1))
    return f


def _load_mesh(xyz, normals, weights, idx):
    u, v = _tangent_vectors(normals)
    dx = xyz[idx] - xyz[:, None, :]
    d2 = (dx ** 2).sum(-1)
    w = jnp.exp(-d2 / (2.0 * RADIUS * RADIUS)) * weights[idx][..., 0]
    o = (w[..., None] * dx).sum(1)
    tu = (o * u).sum(-1)
    tv = (o * v).sum(-1)
    nrm = jnp.sqrt(tu * tu + tv * tv + 1e-8)
    tu = tu / nrm
    tv = tv / nrm
    u2 = tu[:, None] * u + tv[:, None] * v
    v2 = -tv[:, None] * u + tu[:, None] * v
    return jnp.stack([normals, u2, v2], axis=1)


def _dmasif_conv(lp, f, xyz, nuv, idx):
    f1 = f @ lp['in_W'] + lp['in_b']
    dx = xyz[idx] - xyz[:, None, :]
    d2 = (dx ** 2).sum(-1)
    ni = nuv[:, 0, :]
    cos = (ni[:, None, :] * ni[idx]).sum(-1)
    d2g = d2 * (2.0 - cos) ** 2
    w = jnp.exp(-d2g / (2.0 * RADIUS * RADIUS))
    X = jnp.einsum('nij,nkj->nki', nuv, dx) / RADIUS
    g = jax.nn.relu(X @ lp['f_W1'] + lp['f_b1']) @ lp['f_W2'] + lp['f_b2']
    out = (w[..., None] * g * f1[idx]).sum(1) / (w.sum(1, keepdims=True) + 1e-8)
    return out @ lp['out_W'] + lp['out_b']


def _mix_kernel(x_ref, w1_ref, b1_ref, w2_ref, b2_ref, wl_ref, bl_ref, f_ref, o_ref):
    x = x_ref[...]
    h = jnp.maximum(x @ w1_ref[...] + b1_ref[...], 0.0)
    o_ref[...] = (h @ w2_ref[...] + b2_ref[...]
                  + f_ref[...] @ wl_ref[...] + bl_ref[...])


def _mix(lp, xi, x):
    return pl.pallas_call(
        _mix_kernel,
        out_shape=jax.ShapeDtypeStruct((x.shape[0], EMB), jnp.float32),
    )(xi, lp['mlp_W1'], lp['mlp_b1'][None], lp['mlp_W2'], lp['mlp_b2'][None],
      lp['lt_W'], lp['lt_b'][None], x)


def _conv_seg(layers, feats, xyz, nuv, idx):
    x = feats
    for lp in layers:
        xi = _dmasif_conv(lp, x, xyz, nuv, idx)
        x = _mix(lp, xi, x)
    return x


def _site_embed(p, xyz, normals, feats, idx):
    s = _lrelu(feats @ p['or_W1'] + p['or_b1']) @ p['or_W2'] + p['or_b2']
    nuv = _load_mesh(xyz, normals, s, idx)
    return _conv_seg(p['conv'], feats, xyz, nuv, idx)


def kernel(surface_xyz, surface_normals, atom_coords, atom_types, params):
    idx_ss = _knn_idx(surface_xyz, surface_xyz, K + 1)[:, 1:]
    idx_aa = _knn_idx(atom_coords, atom_coords, K + 1)[:, 1:]
    idx_pa = _knn_idx(surface_xyz, atom_coords, K)
    curv = _curvature_features(surface_xyz, surface_normals, idx_ss)
    chem = _atomnet(params['atomnet'], surface_xyz, atom_coords, atom_types, idx_aa, idx_pa)
    in_feats = jnp.concatenate([curv, chem], axis=1)
    out_feats = _site_embed(params['embed'], surface_xyz, surface_normals, in_feats, idx_ss)
    return in_feats, out_feats


# speedup vs baseline: 7.3988x; 3.2942x over previous
import jax
import jax.numpy as jnp
from jax.experimental import pallas as pl

N = 8192
A = 2048
D_ATOM = 6
K = 16
SCALES = [1.0, 2.0, 3.0, 5.0, 10.0]
RADIUS = 9.0
IN_CH = 2 * len(SCALES) + D_ATOM
EMB = 64


def _lrelu(x):
    return jax.nn.leaky_relu(x, 0.2)


def _knn_kernel(nr, exclude_self, q_ref, rT_ref, idx_ref):
    bq = q_ref.shape[0]
    rT = rT_ref[...]
    r2 = (rT * rT).sum(0, keepdims=True)
    s = r2
    for c in range(3):
        s = s - (2.0 * q_ref[:, c:c + 1]) * rT[c:c + 1, :]
    col = jax.lax.broadcasted_iota(jnp.int32, (bq, nr), 1)
    if exclude_self:
        row0 = pl.program_id(0) * bq
        row = row0 + jax.lax.broadcasted_iota(jnp.int32, (bq, nr), 0)
        s = jnp.where(col == row, jnp.inf, s)
    cols = []
    for _ in range(K):
        m = s.min(axis=1, keepdims=True)
        cand = jnp.where(s == m, col, nr)
        a = cand.min(axis=1, keepdims=True)
        cols.append(a)
        s = jnp.where(col == a, jnp.inf, s)
    idx_ref[...] = jnp.concatenate(cols, axis=1)


def _knn_idx(q, r, exclude_self, block=256):
    import functools
    nq = q.shape[0]
    nr = r.shape[0]
    rT = r.T
    return pl.pallas_call(
        functools.partial(_knn_kernel, nr, exclude_self),
        grid=(nq // block,),
        in_specs=[
            pl.BlockSpec((block, 3), lambda i: (i, 0)),
            pl.BlockSpec((3, nr), lambda i: (0, 0)),
        ],
        out_specs=pl.BlockSpec((block, K), lambda i: (i, 0)),
        out_shape=jax.ShapeDtypeStruct((nq, K), jnp.int32),
    )(q, rT)


def _curv_kernel(xyzT_ref, nrmT_ref, xjT_ref, njT_ref,
                 curv_ref, d2_ref, cos_ref, dx_ref):
    xyzT = xyzT_ref[...]
    nrmT = nrmT_ref[...]
    xj = xjT_ref[...]
    nj = njT_ref[...]
    dx = xj - xyzT[:, None, :]
    d2 = (dx * dx).sum(0)
    dx_ref[...] = dx
    d2_ref[...] = d2
    cos_ref[...] = (nrmT[:, None, :] * nj).sum(0)
    rows = []
    for s in SCALES:
        w = jnp.exp(d2 * (-1.0 / (2.0 * s * s)))
        ns = (w[None] * nj).sum(1) + nrmT
        nslen = jnp.sqrt((ns * ns).sum(0, keepdims=True))
        ns = ns / (nslen + 1e-8)
        nx, ny, nz = ns[0:1], ns[1:2], ns[2:3]
        sg = jnp.where(nz >= 0, 1.0, -1.0)
        a = -1.0 / (sg + nz)
        b = nx * ny * a
        u = jnp.concatenate([1.0 + sg * nx * nx * a, sg * b, -sg * nx], axis=0)
        v = jnp.concatenate([b, sg + ny * ny * a, -ny], axis=0)
        P0 = (u[:, None, :] * dx).sum(0)
        P1 = (v[:, None, :] * dx).sum(0)
        Q0 = (u[:, None, :] * nj).sum(0)
        Q1 = (v[:, None, :] * nj).sum(0)
        wP0 = w * P0
        wP1 = w * P1
        ppt00 = (wP0 * P0).sum(0, keepdims=True) + 0.01
        ppt01 = (wP0 * P1).sum(0, keepdims=True)
        ppt11 = (wP1 * P1).sum(0, keepdims=True) + 0.01
        pqt00 = (wP0 * Q0).sum(0, keepdims=True)
        pqt01 = (wP0 * Q1).sum(0, keepdims=True)
        pqt10 = (wP1 * Q0).sum(0, keepdims=True)
        pqt11 = (wP1 * Q1).sum(0, keepdims=True)
        det = ppt00 * ppt11 - ppt01 * ppt01
        s00 = (ppt11 * pqt00 - ppt01 * pqt10) / det
        s01 = (ppt11 * pqt01 - ppt01 * pqt11) / det
        s10 = (ppt00 * pqt10 - ppt01 * pqt00) / det
        s11 = (ppt00 * pqt11 - ppt01 * pqt01) / det
        rows.append(jnp.clip(s00 + s11, -1.0, 1.0))
        rows.append(jnp.clip(s00 * s11 - s01 * s10, -1.0, 1.0))
    curv_ref[...] = jnp.concatenate(rows, axis=0)


def _curvature_pallas(xyz, normals, idx):
    xyzT = xyz.T
    nrmT = normals.T
    xjT = xyz[idx].transpose(2, 1, 0)
    njT = normals[idx].transpose(2, 1, 0)
    n = xyz.shape[0]
    curvT, d2T, cosT, dxT = pl.pallas_call(
        _curv_kernel,
        out_shape=[
            jax.ShapeDtypeStruct((2 * len(SCALES), n), jnp.float32),
            jax.ShapeDtypeStruct((K, n), jnp.float32),
            jax.ShapeDtypeStruct((K, n), jnp.float32),
            jax.ShapeDtypeStruct((3, K, n), jnp.float32),
        ],
    )(xyzT, nrmT, xjT, njT)
    return curvT, d2T, cosT, dxT


def _curvature_features(xyz, normals, idx):
    return _curvature_pallas(xyz, normals, idx)[0].T


def _tangent_vectors(n):
    x, y, z = n[:, 0], n[:, 1], n[:, 2]
    s = jnp.where(z >= 0, 1.0, -1.0)
    a = -1.0 / (s + z)
    b = x * y * a
    u = jnp.stack([1.0 + s * x * x * a, s * b, -s * x], axis=-1)
    v = jnp.stack([b, s + y * y * a, -y], axis=-1)
    return u, v


def _atomnet(p, xyz, atom_xyz, atom_types, idx_aa, idx_pa):
    t = _lrelu(atom_types @ p['tt_W1'] + p['tt_b1']) @ p['tt_W2'] + p['tt_b2']
    d_aa = jnp.linalg.norm(atom_xyz[idx_aa] - atom_xyz[:, None, :], axis=-1, keepdims=True)
    y = t
    for i in range(3):
        h = jnp.concatenate([y[idx_aa], d_aa], axis=-1)
        m = _lrelu(h @ p['aa_W1'][i] + p['aa_b1'][i]) @ p['aa_W2'][i] + p['aa_b2'][i]
        y = y + jax.nn.relu(m.mean(axis=1))
    d_pa = jnp.linalg.norm(atom_xyz[idx_pa] - xyz[:, None, :], axis=-1, keepdims=True)
    fj = y[idx_pa]
    f = fj.mean(axis=1)
    for i in range(3):
        h = jnp.concatenate([fj, d_pa], axis=-1)
        m = _lrelu(h @ p['pa_W1'][i] + p['pa_b1'][i]) @ p['pa_W2'][i] + p['pa_b2'][i]
        f = f + jax.nn.relu(m.mean(axis=1))
    return f


def _load_mesh(xyz, normals, weights, idx):
    u, v = _tangent_vectors(normals)
    dx = xyz[idx] - xyz[:, None, :]
    d2 = (dx ** 2).sum(-1)
    w = jnp.exp(-d2 / (2.0 * RADIUS * RADIUS)) * weights[idx][..., 0]
    o = (w[..., None] * dx).sum(1)
    tu = (o * u).sum(-1)
    tv = (o * v).sum(-1)
    nrm = jnp.sqrt(tu * tu + tv * tv + 1e-8)
    tu = tu / nrm
    tv = tv / nrm
    u2 = tu[:, None] * u + tv[:, None] * v
    v2 = -tv[:, None] * u + tu[:, None] * v
    return jnp.stack([normals, u2, v2], axis=1)


def _dmasif_conv(lp, f, xyz, nuv, idx):
    f1 = f @ lp['in_W'] + lp['in_b']
    dx = xyz[idx] - xyz[:, None, :]
    d2 = (dx ** 2).sum(-1)
    ni = nuv[:, 0, :]
    cos = (ni[:, None, :] * ni[idx]).sum(-1)
    d2g = d2 * (2.0 - cos) ** 2
    w = jnp.exp(-d2g / (2.0 * RADIUS * RADIUS))
    X = jnp.einsum('nij,nkj->nki', nuv, dx) / RADIUS
    g = jax.nn.relu(X @ lp['f_W1'] + lp['f_b1']) @ lp['f_W2'] + lp['f_b2']
    out = (w[..., None] * g * f1[idx]).sum(1) / (w.sum(1, keepdims=True) + 1e-8)
    return out @ lp['out_W'] + lp['out_b']


def _mix_kernel(x_ref, w1_ref, b1_ref, w2_ref, b2_ref, wl_ref, bl_ref, f_ref, o_ref):
    x = x_ref[...]
    h = jnp.maximum(x @ w1_ref[...] + b1_ref[...], 0.0)
    o_ref[...] = (h @ w2_ref[...] + b2_ref[...]
                  + f_ref[...] @ wl_ref[...] + bl_ref[...])


def _mix(lp, xi, x):
    return pl.pallas_call(
        _mix_kernel,
        out_shape=jax.ShapeDtypeStruct((x.shape[0], EMB), jnp.float32),
    )(xi, lp['mlp_W1'], lp['mlp_b1'][None], lp['mlp_W2'], lp['mlp_b2'][None],
      lp['lt_W'], lp['lt_b'][None], x)


def _conv_seg(layers, feats, xyz, nuv, idx):
    x = feats
    for lp in layers:
        xi = _dmasif_conv(lp, x, xyz, nuv, idx)
        x = _mix(lp, xi, x)
    return x


def _site_embed(p, xyz, normals, feats, idx):
    s = _lrelu(feats @ p['or_W1'] + p['or_b1']) @ p['or_W2'] + p['or_b2']
    nuv = _load_mesh(xyz, normals, s, idx)
    return _conv_seg(p['conv'], feats, xyz, nuv, idx)


def kernel(surface_xyz, surface_normals, atom_coords, atom_types, params):
    idx_ss = _knn_idx(surface_xyz, surface_xyz, True)
    idx_aa = _knn_idx(atom_coords, atom_coords, True)
    idx_pa = _knn_idx(surface_xyz, atom_coords, False)
    curv = _curvature_features(surface_xyz, surface_normals, idx_ss)
    chem = _atomnet(params['atomnet'], surface_xyz, atom_coords, atom_types, idx_aa, idx_pa)
    in_feats = jnp.concatenate([curv, chem], axis=1)
    out_feats = _site_embed(params['embed'], surface_xyz, surface_normals, in_feats, idx_ss)
    return in_feats, out_feats


# speedup vs baseline: 29.4990x; 3.9870x over previous
import functools

import jax
import jax.numpy as jnp
from jax import lax
from jax.experimental import pallas as pl
from jax.experimental.pallas import tpu as pltpu
from jax.experimental.pallas import tpu_sc as plsc

N = 8192
A = 2048
D_ATOM = 6
K = 16
SCALES = [1.0, 2.0, 3.0, 5.0, 10.0]
RADIUS = 9.0
IN_CH = 2 * len(SCALES) + D_ATOM
EMB = 64

_NC, _NS, _L = 2, 16, 16
_NW = _NC * _NS


def _sc_mesh():
    return plsc.VectorSubcoreMesh(core_axis_name="c", subcore_axis_name="s")


def _sc_gather_cols(tableT, idx):
    D, V = tableT.shape
    B = idx.shape[0]
    bpw = B // _NW
    cw = min(bpw, 2048)
    nslab = bpw // cw

    @functools.partial(
        pl.kernel,
        out_type=jax.ShapeDtypeStruct((D * B,), jnp.float32),
        mesh=_sc_mesh(),
        scratch_types=[
            pltpu.VMEM((D * V,), jnp.float32),
            pltpu.VMEM((cw,), jnp.int32),
            pltpu.VMEM((D * cw,), jnp.float32),
        ],
        compiler_params=pltpu.CompilerParams(needs_layout_passes=False),
    )
    def k(tbl_hbm, idx_hbm, out_hbm, tbl_v, idx_v, out_v):
        wid = lax.axis_index("s") * _NC + lax.axis_index("c")
        base = wid * bpw
        pltpu.sync_copy(tbl_hbm, tbl_v)

        def slab(t, carry):
            off = base + t * cw
            pltpu.sync_copy(idx_hbm.at[pl.ds(off, cw)], idx_v)

            def body(j, c2):
                iv = idx_v[pl.ds(j * _L, _L)]
                for c in range(D):
                    out_v[pl.ds(c * cw + j * _L, _L)] = plsc.load_gather(
                        tbl_v, [iv + c * V])
                return c2

            lax.fori_loop(0, cw // _L, body, 0)
            for c in range(D):
                pltpu.sync_copy(out_v.at[pl.ds(c * cw, cw)],
                                out_hbm.at[pl.ds(c * B + off, cw)])
            return carry

        lax.fori_loop(0, nslab, slab, 0)

    return k(tableT.reshape(-1), idx).reshape(D, B)


def _sc_gather_rows(table, idx, chunk=512):
    V, D = table.shape
    B = idx.shape[0]
    bpw = B // _NW
    nchunk = bpw // chunk

    @functools.partial(
        pl.kernel,
        out_type=jax.ShapeDtypeStruct((B, D), jnp.float32),
        mesh=_sc_mesh(),
        scratch_types=[
            pltpu.VMEM((bpw,), jnp.int32),
            pltpu.VMEM((chunk, D), jnp.float32),
            pltpu.SemaphoreType.DMA,
        ],
    )
    def k(tbl_hbm, idx_hbm, out_hbm, idx_v, rows_v, sem):
        wid = lax.axis_index("s") * _NC + lax.axis_index("c")
        base = wid * bpw
        pltpu.sync_copy(idx_hbm.at[pl.ds(base, bpw)], idx_v)
        for t in range(nchunk):
            pltpu.async_copy(
                tbl_hbm.at[idx_v.at[pl.ds(t * chunk, chunk)]], rows_v, sem
            ).wait()
            pltpu.sync_copy(rows_v, out_hbm.at[pl.ds(base + t * chunk, chunk)])

    return k(table, idx)


_NCH = 18


def _knn_score_kernel(nq, exclude_self, r_ref, qT_ref, m_ref):
    bq = qT_ref.shape[1]
    rr = r_ref[...]
    br = rr.shape[0]
    r2 = (rr * rr).sum(1, keepdims=True)
    qT = qT_ref[...]
    s = r2 - 2.0 * (rr[:, 0:1] * qT[0:1, :] + rr[:, 1:2] * qT[1:2, :]
                    + rr[:, 2:3] * qT[2:3, :])
    if exclude_self:
        rowc = pl.program_id(0) * br + jax.lax.broadcasted_iota(
            jnp.int32, (br, bq), 0)
        colq = pl.program_id(1) * bq + jax.lax.broadcasted_iota(
            jnp.int32, (br, bq), 1)
        s = jnp.where(rowc == colq, jnp.inf, s)
    m_ref[...] = s.reshape(br // 16, 16, bq).min(axis=1)


def _knn_chunk_rank_kernel(nc, m_ref, elem_ref):
    bq = m_ref.shape[1]
    m = m_ref[...]
    rowi = jax.lax.broadcasted_iota(jnp.int32, (nc, bq), 0)
    picks = []
    for _ in range(_NCH):
        mn = m.min(axis=0, keepdims=True)
        cand = jnp.where(m == mn, rowi, nc)
        a = cand.min(axis=0, keepdims=True)
        picks.append(a)
        m = jnp.where(rowi == a, jnp.inf, m)
    rows = []
    for a in picks:
        for l in range(16):
            rows.append(a * 16 + l)
    elem_ref[...] = jnp.concatenate(rows, axis=0)


def _knn_refine_kernel(nr, exclude_self, rg_ref, elem_ref, qT_ref, idx_ref):
    bq = qT_ref.shape[1]
    rg = rg_ref[...]
    elem = elem_ref[...]
    qT = qT_ref[...]
    s = rg[3] - 2.0 * (rg[0] * qT[0:1, :] + rg[1] * qT[1:2, :]
                       + rg[2] * qT[2:3, :])
    if exclude_self:
        colq = pl.program_id(0) * bq + jax.lax.broadcasted_iota(
            jnp.int32, (_NCH * 16, bq), 1)
        s = jnp.where(elem == colq, jnp.inf, s)
    cols = []
    for _ in range(K):
        mn = s.min(axis=0, keepdims=True)
        cand = jnp.where(s == mn, elem, nr)
        a = cand.min(axis=0, keepdims=True)
        cols.append(a)
        s = jnp.where(elem == a, jnp.inf, s)
    idx_ref[...] = jnp.concatenate(cols, axis=0)


def _r4T_kernel(rT_ref, o_ref):
    rT = rT_ref[...]
    r2 = (rT * rT).sum(0, keepdims=True)
    o_ref[...] = jnp.concatenate([rT, r2], axis=0)


def _knn_idx_T(q, r, exclude_self, br_a=2048, bq_a=256, bq_b=256, bq_c=512):
    nq = q.shape[0]
    nr = r.shape[0]
    nc = nr // 16
    br_a = min(br_a, nr)
    qT = q.T
    mT = pl.pallas_call(
        functools.partial(_knn_score_kernel, nq, exclude_self),
        grid=(nr // br_a, nq // bq_a),
        in_specs=[
            pl.BlockSpec((br_a, 3), lambda j, i: (j, 0)),
            pl.BlockSpec((3, bq_a), lambda j, i: (0, i)),
        ],
        out_specs=pl.BlockSpec((br_a // 16, bq_a), lambda j, i: (j, i)),
        out_shape=jax.ShapeDtypeStruct((nc, nq), jnp.float32),
    )(r, qT)
    elemT = pl.pallas_call(
        functools.partial(_knn_chunk_rank_kernel, nc),
        grid=(nq // bq_b,),
        in_specs=[
            pl.BlockSpec((nc, bq_b), lambda i: (0, i)),
        ],
        out_specs=pl.BlockSpec((_NCH * 16, bq_b), lambda i: (0, i)),
        out_shape=jax.ShapeDtypeStruct((_NCH * 16, nq), jnp.int32),
    )(mT)
    r4T = pl.pallas_call(
        _r4T_kernel,
        out_shape=jax.ShapeDtypeStruct((4, nr), jnp.float32),
    )(r.T)
    rg = _sc_gather_cols(r4T, elemT.reshape(-1)).reshape(4, _NCH * 16, nq)
    idxT = pl.pallas_call(
        functools.partial(_knn_refine_kernel, nr, exclude_self),
        grid=(nq // bq_c,),
        in_specs=[
            pl.BlockSpec((4, _NCH * 16, bq_c), lambda i: (0, 0, i)),
            pl.BlockSpec((_NCH * 16, bq_c), lambda i: (0, i)),
            pl.BlockSpec((3, bq_c), lambda i: (0, i)),
        ],
        out_specs=pl.BlockSpec((K, bq_c), lambda i: (0, i)),
        out_shape=jax.ShapeDtypeStruct((K, nq), jnp.int32),
    )(rg, elemT, qT)
    return idxT


def _knn_kernel(nr, exclude_self, q_ref, rT_ref, idx_ref):
    bq = q_ref.shape[0]
    rT = rT_ref[...]
    r2 = (rT * rT).sum(0, keepdims=True)
    s = r2
    for c in range(3):
        s = s - (2.0 * q_ref[:, c:c + 1]) * rT[c:c + 1, :]
    col = jax.lax.broadcasted_iota(jnp.int32, (bq, nr), 1)
    if exclude_self:
        row0 = pl.program_id(0) * bq
        row = row0 + jax.lax.broadcasted_iota(jnp.int32, (bq, nr), 0)
        s = jnp.where(col == row, jnp.inf, s)
    cols = []
    for _ in range(K):
        m = s.min(axis=1, keepdims=True)
        cand = jnp.where(s == m, col, nr)
        a = cand.min(axis=1, keepdims=True)
        cols.append(a)
        s = jnp.where(col == a, jnp.inf, s)
    idx_ref[...] = jnp.concatenate(cols, axis=1)


def _knn_idx(q, r, exclude_self, block=256):
    nq = q.shape[0]
    nr = r.shape[0]
    rT = r.T
    return pl.pallas_call(
        functools.partial(_knn_kernel, nr, exclude_self),
        grid=(nq // block,),
        in_specs=[
            pl.BlockSpec((block, 3), lambda i: (i, 0)),
            pl.BlockSpec((3, nr), lambda i: (0, 0)),
        ],
        out_specs=pl.BlockSpec((block, K), lambda i: (i, 0)),
        out_shape=jax.ShapeDtypeStruct((nq, K), jnp.int32),
    )(q, rT)


def _curv_kernel(xyzT_ref, nrmT_ref, xjT_ref, njT_ref,
                 curv_ref, d2_ref, cos_ref, dx_ref):
    xyzT = xyzT_ref[...]
    nrmT = nrmT_ref[...]
    xj = xjT_ref[...]
    nj = njT_ref[...]
    dx = xj - xyzT[:, None, :]
    d2 = (dx * dx).sum(0)
    dx_ref[...] = dx
    d2_ref[...] = d2
    cos_ref[...] = (nrmT[:, None, :] * nj).sum(0)
    rows = []
    for s in SCALES:
        w = jnp.exp(d2 * (-1.0 / (2.0 * s * s)))
        ns = (w[None] * nj).sum(1) + nrmT
        nslen = jnp.sqrt((ns * ns).sum(0, keepdims=True))
        ns = ns / (nslen + 1e-8)
        nx, ny, nz = ns[0:1], ns[1:2], ns[2:3]
        sg = jnp.where(nz >= 0, 1.0, -1.0)
        a = -1.0 / (sg + nz)
        b = nx * ny * a
        u = jnp.concatenate([1.0 + sg * nx * nx * a, sg * b, -sg * nx], axis=0)
        v = jnp.concatenate([b, sg + ny * ny * a, -ny], axis=0)
        P0 = (u[:, None, :] * dx).sum(0)
        P1 = (v[:, None, :] * dx).sum(0)
        Q0 = (u[:, None, :] * nj).sum(0)
        Q1 = (v[:, None, :] * nj).sum(0)
        wP0 = w * P0
        wP1 = w * P1
        ppt00 = (wP0 * P0).sum(0, keepdims=True) + 0.01
        ppt01 = (wP0 * P1).sum(0, keepdims=True)
        ppt11 = (wP1 * P1).sum(0, keepdims=True) + 0.01
        pqt00 = (wP0 * Q0).sum(0, keepdims=True)
        pqt01 = (wP0 * Q1).sum(0, keepdims=True)
        pqt10 = (wP1 * Q0).sum(0, keepdims=True)
        pqt11 = (wP1 * Q1).sum(0, keepdims=True)
        det = ppt00 * ppt11 - ppt01 * ppt01
        s00 = (ppt11 * pqt00 - ppt01 * pqt10) / det
        s01 = (ppt11 * pqt01 - ppt01 * pqt11) / det
        s10 = (ppt00 * pqt10 - ppt01 * pqt00) / det
        s11 = (ppt00 * pqt11 - ppt01 * pqt01) / det
        rows.append(jnp.clip(s00 + s11, -1.0, 1.0))
        rows.append(jnp.clip(s00 * s11 - s01 * s10, -1.0, 1.0))
    curv_ref[...] = jnp.concatenate(rows, axis=0)


def _curvature_pallas(xyzT, nrmT, xjT, njT, bn=1024):
    return pl.pallas_call(
        _curv_kernel,
        grid=(N // bn,),
        in_specs=[
            pl.BlockSpec((3, bn), lambda i: (0, i)),
            pl.BlockSpec((3, bn), lambda i: (0, i)),
            pl.BlockSpec((3, K, bn), lambda i: (0, 0, i)),
            pl.BlockSpec((3, K, bn), lambda i: (0, 0, i)),
        ],
        out_specs=[
            pl.BlockSpec((2 * len(SCALES), bn), lambda i: (0, i)),
            pl.BlockSpec((K, bn), lambda i: (0, i)),
            pl.BlockSpec((K, bn), lambda i: (0, i)),
            pl.BlockSpec((3, K, bn), lambda i: (0, 0, i)),
        ],
        out_shape=[
            jax.ShapeDtypeStruct((2 * len(SCALES), N), jnp.float32),
            jax.ShapeDtypeStruct((K, N), jnp.float32),
            jax.ShapeDtypeStruct((K, N), jnp.float32),
            jax.ShapeDtypeStruct((3, K, N), jnp.float32),
        ],
    )(xyzT, nrmT, xjT, njT)


def _atom_t_kernel(atT_ref, w1T_ref, b1_ref, w2T_ref, b2_ref, o_ref):
    h = w1T_ref[...] @ atT_ref[...] + b1_ref[...]
    h = jnp.where(h >= 0, h, 0.2 * h)
    o_ref[...] = w2T_ref[...] @ h + b2_ref[...]


def _dT_kernel(cgT_ref, xT_ref, d_ref):
    cg = cgT_ref[...]
    dx = cg - xT_ref[...][:, None, :]
    d_ref[...] = jnp.sqrt((dx * dx).sum(0))


def _mlp_round(h, w1, b1, w2, b2):
    n_in = len(h)
    n_hid = w1.shape[1]
    n_out = w2.shape[1]
    hid = []
    for o in range(n_hid):
        acc = b1[o]
        for c in range(n_in):
            acc = acc + w1[c, o] * h[c]
        hid.append(jnp.where(acc >= 0, acc, 0.2 * acc))
    outs = []
    for o2 in range(n_out):
        acc = b2[o2]
        for o in range(n_hid):
            acc = acc + w2[o, o2] * hid[o]
        mm = acc.sum(0, keepdims=True) * (1.0 / K)
        outs.append(jnp.maximum(mm, 0.0))
    return outs


def _aa_round_kernel(yT_ref, ygT_ref, dT_ref, w1_ref, b1_ref, w2_ref, b2_ref,
                     o_ref):
    yg = ygT_ref[...]
    h = [yg[c] for c in range(D_ATOM)] + [dT_ref[...]]
    outs = _mlp_round(h, w1_ref[...], b1_ref[...][0], w2_ref[...],
                      b2_ref[...][0])
    o_ref[...] = yT_ref[...] + jnp.concatenate(outs, axis=0)


def _pa_kernel(fjT_ref, dT_ref, w1s_ref, b1s_ref, w2s_ref, b2s_ref, o_ref):
    fj = fjT_ref[...]
    h = [fj[c] for c in range(D_ATOM)] + [dT_ref[...]]
    f = [h[c].sum(0, keepdims=True) * (1.0 / K) for c in range(D_ATOM)]
    for i in range(3):
        outs = _mlp_round(h, w1s_ref[i], b1s_ref[i][0], w2s_ref[i],
                          b2s_ref[i][0])
        f = [a + b for a, b in zip(f, outs)]
    o_ref[...] = jnp.concatenate(f, axis=0)


def _atomnet_T(p, xyzT, axT, atT, idx_aaT, idx_paT):
    tT = pl.pallas_call(
        _atom_t_kernel,
        out_shape=jax.ShapeDtypeStruct((D_ATOM, A), jnp.float32),
    )(atT, p['tt_W1'].T, p['tt_b1'][:, None], p['tt_W2'].T, p['tt_b2'][:, None])

    cgT = _sc_gather_cols(axT, idx_aaT).reshape(3, K, A)
    d_aaT = pl.pallas_call(
        _dT_kernel,
        out_shape=jax.ShapeDtypeStruct((K, A), jnp.float32),
    )(cgT, axT)

    yT = tT
    for i in range(3):
        ygT = _sc_gather_cols(yT, idx_aaT).reshape(D_ATOM, K, A)
        yT = pl.pallas_call(
            _aa_round_kernel,
            out_shape=jax.ShapeDtypeStruct((D_ATOM, A), jnp.float32),
        )(yT, ygT, d_aaT, p['aa_W1'][i], p['aa_b1'][i][None],
          p['aa_W2'][i], p['aa_b2'][i][None])

    apT = _sc_gather_cols(axT, idx_paT).reshape(3, K, N)
    d_paT = pl.pallas_call(
        _dT_kernel,
        out_shape=jax.ShapeDtypeStruct((K, N), jnp.float32),
    )(apT, xyzT)
    fjT = _sc_gather_cols(yT, idx_paT).reshape(D_ATOM, K, N)
    bn = 1024
    chemT = pl.pallas_call(
        _pa_kernel,
        grid=(N // bn,),
        in_specs=[
            pl.BlockSpec((D_ATOM, K, bn), lambda i: (0, 0, i)),
            pl.BlockSpec((K, bn), lambda i: (0, i)),
            pl.BlockSpec((3, D_ATOM + 1, 2 * D_ATOM), lambda i: (0, 0, 0)),
            pl.BlockSpec((3, 1, 2 * D_ATOM), lambda i: (0, 0, 0)),
            pl.BlockSpec((3, 2 * D_ATOM, D_ATOM), lambda i: (0, 0, 0)),
            pl.BlockSpec((3, 1, D_ATOM), lambda i: (0, 0, 0)),
        ],
        out_specs=pl.BlockSpec((D_ATOM, bn), lambda i: (0, i)),
        out_shape=jax.ShapeDtypeStruct((D_ATOM, N), jnp.float32),
    )(fjT, d_paT, p['pa_W1'], p['pa_b1'][:, None, :],
      p['pa_W2'], p['pa_b2'][:, None, :])
    return chemT


def _or_kernel(f_ref, w1_ref, b1_ref, w2_ref, b2_ref, o_ref):
    h = f_ref[...] @ w1_ref[...] + b1_ref[...]
    h = jnp.where(h >= 0, h, 0.2 * h)
    o_ref[...] = h @ w2_ref[...] + b2_ref[...]


def _mesh_kernel(nrmT_ref, dx_ref, d2_ref, cos_ref, sg_ref,
                 XT_ref, w_ref, wsum_ref):
    nrmT = nrmT_ref[...]
    dx = dx_ref[...]
    d2 = d2_ref[...]
    nx, ny, nz = nrmT[0:1], nrmT[1:2], nrmT[2:3]
    sg = jnp.where(nz >= 0, 1.0, -1.0)
    a = -1.0 / (sg + nz)
    b = nx * ny * a
    u = jnp.concatenate([1.0 + sg * nx * nx * a, sg * b, -sg * nx], axis=0)
    v = jnp.concatenate([b, sg + ny * ny * a, -ny], axis=0)
    wm = jnp.exp(d2 * (-1.0 / (2.0 * RADIUS * RADIUS))) * sg_ref[...]
    o = (wm[None] * dx).sum(1)
    tu = (o * u).sum(0, keepdims=True)
    tv = (o * v).sum(0, keepdims=True)
    nrm = jnp.sqrt(tu * tu + tv * tv + 1e-8)
    tu = tu / nrm
    tv = tv / nrm
    u2 = tu * u + tv * v
    v2 = -tv * u + tu * v
    inv_r = 1.0 / RADIUS
    x0 = (nrmT[:, None, :] * dx).sum(0) * inv_r
    x1 = (u2[:, None, :] * dx).sum(0) * inv_r
    x2 = (v2[:, None, :] * dx).sum(0) * inv_r
    XT_ref[...] = jnp.stack([x0, x1, x2], axis=0)
    cosv = cos_ref[...]
    q = 2.0 - cosv
    wc = jnp.exp(d2 * q * q * (-1.0 / (2.0 * RADIUS * RADIUS)))
    w_ref[...] = wc
    wsum_ref[...] = wc.sum(0, keepdims=True)


def _f1_kernel(x_ref, w_ref, b_ref, o_ref):
    f1 = x_ref[...] @ w_ref[...] + b_ref[...]
    o_ref[...] = jnp.concatenate([f1, jnp.zeros_like(f1)], axis=1)


def _conv_block_kernel(X_ref, f1g_ref, w_ref, wsum_ref, x_ref,
                       fw1_ref, fb1_ref, fw2_ref, fb2_ref,
                       ow_ref, ob_ref, mw1_ref, mb1_ref, mw2_ref, mb2_ref,
                       lw_ref, lb_ref, o_ref):
    bn = x_ref.shape[0]
    g = jnp.maximum(X_ref[...] @ fw1_ref[...] + fb1_ref[...], 0.0)
    g = g @ fw2_ref[...] + fb2_ref[...]
    prod = (g * f1g_ref[...]).reshape(bn, K, EMB) * w_ref[...]
    s = prod.sum(1)
    o = s / (wsum_ref[...] + 1e-8)
    o = o @ ow_ref[...] + ob_ref[...]
    xi = jnp.maximum(o @ mw1_ref[...] + mb1_ref[...], 0.0)
    xi = xi @ mw2_ref[...] + mb2_ref[...]
    o_ref[...] = xi + x_ref[...] @ lw_ref[...] + lb_ref[...]


def _conv_layer(lp, x, Xn, w, wsum, idxN, nblk=256):
    ich = x.shape[1]
    f1 = pl.pallas_call(
        _f1_kernel,
        out_shape=jax.ShapeDtypeStruct((N, 2 * EMB), jnp.float32),
    )(x, lp['in_W'], lp['in_b'][None])
    f1g = _sc_gather_rows(f1, idxN)[:, :EMB]
    grid = N // nblk
    return pl.pallas_call(
        _conv_block_kernel,
        grid=(grid,),
        in_specs=[
            pl.BlockSpec((nblk * K, 3), lambda i: (i, 0)),
            pl.BlockSpec((nblk * K, EMB), lambda i: (i, 0)),
            pl.BlockSpec((nblk, K, 1), lambda i: (i, 0, 0)),
            pl.BlockSpec((nblk, 1), lambda i: (i, 0)),
            pl.BlockSpec((nblk, ich), lambda i: (i, 0)),
            pl.BlockSpec((3, 8), lambda i: (0, 0)),
            pl.BlockSpec((1, 8), lambda i: (0, 0)),
            pl.BlockSpec((8, EMB), lambda i: (0, 0)),
            pl.BlockSpec((1, EMB), lambda i: (0, 0)),
            pl.BlockSpec((EMB, EMB), lambda i: (0, 0)),
            pl.BlockSpec((1, EMB), lambda i: (0, 0)),
            pl.BlockSpec((EMB, EMB), lambda i: (0, 0)),
            pl.BlockSpec((1, EMB), lambda i: (0, 0)),
            pl.BlockSpec((EMB, EMB), lambda i: (0, 0)),
            pl.BlockSpec((1, EMB), lambda i: (0, 0)),
            pl.BlockSpec((ich, EMB), lambda i: (0, 0)),
            pl.BlockSpec((1, EMB), lambda i: (0, 0)),
        ],
        out_specs=pl.BlockSpec((nblk, EMB), lambda i: (i, 0)),
        out_shape=jax.ShapeDtypeStruct((N, EMB), jnp.float32),
    )(Xn, f1g, w, wsum, x,
      lp['f_W1'], lp['f_b1'][None], lp['f_W2'], lp['f_b2'][None],
      lp['out_W'], lp['out_b'][None], lp['mlp_W1'], lp['mlp_b1'][None],
      lp['mlp_W2'], lp['mlp_b2'][None], lp['lt_W'], lp['lt_b'][None])


def _site_embed(p, nrmT, feats, idxT, idxN, dxT, d2T, cosT):
    s = pl.pallas_call(
        _or_kernel,
        out_shape=jax.ShapeDtypeStruct((N, 1), jnp.float32),
    )(feats, p['or_W1'], p['or_b1'][None], p['or_W2'], p['or_b2'][None])
    sgT = _sc_gather_cols(s.T, idxT).reshape(K, N)
    bn = 1024
    XT, wT, wsumT = pl.pallas_call(
        _mesh_kernel,
        grid=(N // bn,),
        in_specs=[
            pl.BlockSpec((3, bn), lambda i: (0, i)),
            pl.BlockSpec((3, K, bn), lambda i: (0, 0, i)),
            pl.BlockSpec((K, bn), lambda i: (0, i)),
            pl.BlockSpec((K, bn), lambda i: (0, i)),
            pl.BlockSpec((K, bn), lambda i: (0, i)),
        ],
        out_specs=[
            pl.BlockSpec((3, K, bn), lambda i: (0, 0, i)),
            pl.BlockSpec((K, bn), lambda i: (0, i)),
            pl.BlockSpec((1, bn), lambda i: (0, i)),
        ],
        out_shape=[
            jax.ShapeDtypeStruct((3, K, N), jnp.float32),
            jax.ShapeDtypeStruct((K, N), jnp.float32),
            jax.ShapeDtypeStruct((1, N), jnp.float32),
        ],
    )(nrmT, dxT, d2T, cosT, sgT)
    Xn = XT.transpose(2, 1, 0).reshape(N * K, 3)
    wk = wT.T.reshape(N, K, 1)
    wsum = wsumT.T
    x = feats
    for lp in p['conv']:
        x = _conv_layer(lp, x, Xn, wk, wsum, idxN)
    return x


def kernel(surface_xyz, surface_normals, atom_coords, atom_types, params):
    idx_ssT = _knn_idx_T(surface_xyz, surface_xyz, True)
    idx_aaTm = _knn_idx_T(atom_coords, atom_coords, True)
    idx_paTm = _knn_idx_T(surface_xyz, atom_coords, False)
    idxT = idx_ssT.reshape(-1)
    idxN = idx_ssT.T.reshape(-1)
    idx_aaT = idx_aaTm.reshape(-1)
    idx_paT = idx_paTm.reshape(-1)

    xyzT = surface_xyz.T
    nrmT = surface_normals.T
    axT = atom_coords.T

    surfT = jnp.concatenate([xyzT, nrmT], axis=0)
    g6 = _sc_gather_cols(surfT, idxT).reshape(6, K, N)
    xjT, njT = g6[:3], g6[3:]

    curvT, d2T, cosT, dxT = _curvature_pallas(xyzT, nrmT, xjT, njT)
    chemT = _atomnet_T(params['atomnet'], xyzT, axT, atom_types.T,
                       idx_aaT, idx_paT)
    in_feats = jnp.concatenate([curvT, chemT], axis=0).T
    out_feats = _site_embed(params['embed'], nrmT, in_feats, idxT, idxN,
                            dxT, d2T, cosT)
    return in_feats, out_feats
